# baseline (speedup 1.0000x reference)
import functools

import jax
import jax.numpy as jnp
from jax import lax
from jax.experimental import pallas as pl
from jax.experimental.pallas import tpu as pltpu
from jax.experimental.pallas import tpu_sc as plsc

_N = 10000
_E = 320000
_DIN = 128
_DHID = 64
_DC = 16
_ALPHA = 0.1
_K = 10

_NT = 16
_NC = 2
_C = 512
_NSUB = 20
_EPAD = _NC * _NT * _NSUB * _C
_RT = _N // _NT
_R = _N + 8


def _mlp_block(x_ref, w1_ref, b1_ref, w2_ref, b2_ref, o_ref):
    x = x_ref[...]
    h = jnp.dot(x, w1_ref[...], preferred_element_type=jnp.float32)
    h = jnp.maximum(h + b1_ref[...], 0.0)
    o = jnp.dot(h, w2_ref[...], preferred_element_type=jnp.float32)
    o_ref[...] = o + b2_ref[...]


def _mlp(x, W1, b1, W2, b2):
    return pl.pallas_call(
        _mlp_block,
        grid=(10,),
        in_specs=[
            pl.BlockSpec((1000, _DIN), lambda i: (i, 0)),
            pl.BlockSpec((_DIN, _DHID), lambda i: (0, 0)),
            pl.BlockSpec((1, _DHID), lambda i: (0, 0)),
            pl.BlockSpec((_DHID, _DC), lambda i: (0, 0)),
            pl.BlockSpec((1, _DC), lambda i: (0, 0)),
        ],
        out_specs=pl.BlockSpec((1000, _DC), lambda i: (i, 0)),
        out_shape=jax.ShapeDtypeStruct((_N, _DC), jnp.float32),
    )(x, W1, b1, W2, b2)


def _rsqrt(v):
    i = lax.bitcast_convert_type(v, jnp.int32)
    i = jnp.int32(0x5F3759DF) - lax.shift_right_arithmetic(i, 1)
    y = lax.bitcast_convert_type(i, jnp.float32)
    for _ in range(3):
        y = y * (1.5 - 0.5 * v * y * y)
    return y


_MESH = plsc.VectorSubcoreMesh(core_axis_name="c", subcore_axis_name="s")


@functools.partial(
    pl.kernel,
    mesh=_MESH,
    compiler_params=pltpu.CompilerParams(use_tc_tiling_on_sc=False),
    out_type=(jax.ShapeDtypeStruct((_N * _DC,), jnp.float32),
              jax.ShapeDtypeStruct((_NC, _R, _DC), jnp.float32),
              jax.ShapeDtypeStruct((_NC, _R, _DC), jnp.float32)),
    scratch_types=[
        pltpu.VMEM_SHARED((_R, _DC), jnp.float32),
        pltpu.VMEM_SHARED((_R, _DC), jnp.float32),
        pltpu.VMEM((_NSUB, _C), jnp.int32),
        pltpu.VMEM((_NSUB, _C), jnp.int32),
        pltpu.VMEM((_C, _DC), jnp.float32),
        pltpu.VMEM((_C, _DC), jnp.float32),
        pltpu.VMEM((_RT, _DC), jnp.float32),
        pltpu.VMEM((_RT, _DC), jnp.float32),
        pltpu.VMEM((_RT, _DC), jnp.float32),
        pltpu.VMEM((_RT, _DC), jnp.float32),
        pltpu.VMEM((_RT, _DC), jnp.float32),
        pltpu.VMEM((_RT, _DC), jnp.float32),
        pltpu.VMEM((_RT * _DC,), jnp.float32),
        pltpu.SemaphoreType.DMA,
        pltpu.SemaphoreType.DMA,
        pltpu.SemaphoreType.DMA,
    ],
)
def _appnp(h_hbm, src_hbm, dst_hbm, out_hbm, p1_hbm, p2_hbm, T1, T2, src_v,
           dst_v, ebuf0, ebuf1, rn_nio, rn_h0o, rn_n9, b_t1, b_p, b_g, b_ho,
           semg, sems0, sems1):
    cid = lax.axis_index("c")
    sid = lax.axis_index("s")
    wid = cid * _NT + sid
    ocid = 1 - cid
    r0 = sid * _RT
    f0 = sid * (_RT * _DC)

    zeros16 = jnp.zeros((_DC,), jnp.float32)
    ones16 = jnp.full((_DC,), 1.0, jnp.float32)

    pltpu.sync_copy(src_hbm.at[wid], src_v)
    pltpu.sync_copy(dst_hbm.at[wid], dst_v)

    def _zero_bg(i, carry):
        b_g[i] = zeros16
        return carry

    def _fill_ones(i, carry):
        ebuf0[i] = ones16
        return carry

    lax.fori_loop(0, _RT, _zero_bg, 0)
    lax.fori_loop(0, _C, _fill_ones, 0)

    pltpu.sync_copy(b_g, T1.at[pl.ds(r0, _RT)])
    pltpu.sync_copy(b_g, T2.at[pl.ds(r0, _RT)])
    plsc.subcore_barrier()

    def _deg(j, carry):
        pltpu.sync_copy(ebuf0, T1.at[src_v.at[j]], add=True)
        pltpu.sync_copy(ebuf0, T2.at[dst_v.at[j]], add=True)
        return carry

    lax.fori_loop(0, _NSUB, _deg, 0)
    plsc.subcore_barrier()

    pltpu.sync_copy(T2.at[pl.ds(r0, _RT)], p1_hbm.at[cid, pl.ds(r0, _RT)])
    pltpu.sync_copy(T1.at[pl.ds(r0, _RT)], p2_hbm.at[cid, pl.ds(r0, _RT)])
    plsc.subcore_barrier()

    pltpu.sync_copy(T2.at[pl.ds(r0, _RT)], b_t1)
    pltpu.sync_copy(p1_hbm.at[ocid, pl.ds(r0, _RT)], b_p)

    def _tf1(i, carry):
        ni = _rsqrt(jnp.maximum(b_t1[i] + b_p[i], 1.0))
        rn_n9[i] = (1.0 - _ALPHA) * ni
        return carry

    lax.fori_loop(0, _RT, _tf1, 0)

    pltpu.sync_copy(T1.at[pl.ds(r0, _RT)], b_t1)
    pltpu.sync_copy(p2_hbm.at[ocid, pl.ds(r0, _RT)], b_p)
    pltpu.sync_copy(h_hbm.at[pl.ds(f0, _RT * _DC)], b_ho)

    def _tf2(i, carry):
        no = _rsqrt(jnp.maximum(b_t1[i] + b_p[i], 1.0))
        hh = b_ho[pl.ds(i * _DC, _DC)]
        rn_nio[i] = no * rn_n9[i]
        rn_h0o[i] = _ALPHA * no * hh
        b_t1[i] = no * hh
        return carry

    lax.fori_loop(0, _RT, _tf2, 0)

    pltpu.sync_copy(b_t1, T1.at[pl.ds(r0, _RT)])
    pltpu.sync_copy(b_g, T2.at[pl.ds(r0, _RT)])
    plsc.subcore_barrier()

    def _step(k, carry):
        def _edge2(jj, c2):
            j0 = 2 * jj
            j1 = j0 + 1

            @pl.when(jj > 0)
            def _drain0():
                pltpu.make_async_copy(ebuf0, T2.at[dst_v.at[0]], sems0).wait()

            pltpu.async_copy(T1.at[src_v.at[j0]], ebuf0, semg).wait()
            pltpu.async_copy(ebuf0, T2.at[dst_v.at[j0]], sems0, add=True)

            @pl.when(jj > 0)
            def _drain1():
                pltpu.make_async_copy(ebuf1, T2.at[dst_v.at[0]], sems1).wait()

            pltpu.async_copy(T1.at[src_v.at[j1]], ebuf1, semg).wait()
            pltpu.async_copy(ebuf1, T2.at[dst_v.at[j1]], sems1, add=True)
            return c2

        lax.fori_loop(0, _NSUB // 2, _edge2, 0)
        pltpu.make_async_copy(ebuf0, T2.at[dst_v.at[0]], sems0).wait()
        pltpu.make_async_copy(ebuf1, T2.at[dst_v.at[0]], sems1).wait()
        plsc.subcore_barrier()

        pltpu.sync_copy(T2.at[pl.ds(r0, _RT)], p1_hbm.at[cid, pl.ds(r0, _RT)])
        plsc.subcore_barrier()

        @pl.when(k < _K - 1)
        def _upd():
            pltpu.async_copy(T2.at[pl.ds(r0, _RT)], b_t1, semg).wait()
            pltpu.async_copy(p1_hbm.at[ocid, pl.ds(r0, _RT)], b_p, sems1).wait()
            zh = pltpu.async_copy(b_g, T2.at[pl.ds(r0, _RT)], sems0)

            def _u(i, c2):
                b_t1[i] = (b_t1[i] + b_p[i]) * rn_nio[i] + rn_h0o[i]
                return c2

            lax.fori_loop(0, _RT, _u, 0, unroll=8)
            pltpu.async_copy(b_t1, T1.at[pl.ds(r0, _RT)], sems1).wait()
            zh.wait()

        plsc.subcore_barrier()
        return carry

    lax.fori_loop(0, _K, _step, 0)

    @pl.when(cid == 0)
    def _out():
        pltpu.sync_copy(T2.at[pl.ds(r0, _RT)], b_t1)
        pltpu.sync_copy(p1_hbm.at[1, pl.ds(r0, _RT)], b_p)
        pltpu.sync_copy(h_hbm.at[pl.ds(f0, _RT * _DC)], b_ho)

        def _o(i, carry):
            hh = b_ho[pl.ds(i * _DC, _DC)]
            b_ho[pl.ds(i * _DC, _DC)] = (b_t1[i] + b_p[i]) * rn_n9[i] + _ALPHA * hh
            return carry

        lax.fori_loop(0, _RT, _o, 0)
        pltpu.sync_copy(b_ho, out_hbm.at[pl.ds(f0, _RT * _DC)])


@jax.jit
def kernel(features, edge_index, W1, b1, W2, b2):
    h = _mlp(features, W1, b1.reshape(1, -1), W2, b2.reshape(1, -1))
    src = edge_index[0]
    dst = edge_index[1]
    pad = jnp.full((_EPAD - _E,), _N, jnp.int32)
    src_p = jnp.concatenate([src, pad]).reshape(_NC * _NT, _NSUB, _C)
    dst_p = jnp.concatenate([dst, pad]).reshape(_NC * _NT, _NSUB, _C)
    out, _, _ = _appnp(h.reshape(-1), src_p, dst_p)
    return out.reshape(_N, _DC)


# speedup vs baseline: 1.0356x; 1.0356x over previous
import functools

import jax
import jax.numpy as jnp
from jax import lax
from jax.experimental import pallas as pl
from jax.experimental.pallas import tpu as pltpu
from jax.experimental.pallas import tpu_sc as plsc

_N = 10000
_E = 320000
_DIN = 128
_DHID = 64
_DC = 16
_ALPHA = 0.1
_K = 10

_NT = 16
_NC = 2
_C = 256
_NSUB = 40
_EPAD = _NC * _NT * _NSUB * _C
_RT = _N // _NT
_R = _N + 8


def _mlp_block(x_ref, w1_ref, b1_ref, w2_ref, b2_ref, o_ref):
    x = x_ref[...]
    h = jnp.dot(x, w1_ref[...], preferred_element_type=jnp.float32)
    h = jnp.maximum(h + b1_ref[...], 0.0)
    o = jnp.dot(h, w2_ref[...], preferred_element_type=jnp.float32)
    o_ref[...] = o + b2_ref[...]


def _mlp(x, W1, b1, W2, b2):
    return pl.pallas_call(
        _mlp_block,
        grid=(10,),
        in_specs=[
            pl.BlockSpec((1000, _DIN), lambda i: (i, 0)),
            pl.BlockSpec((_DIN, _DHID), lambda i: (0, 0)),
            pl.BlockSpec((1, _DHID), lambda i: (0, 0)),
            pl.BlockSpec((_DHID, _DC), lambda i: (0, 0)),
            pl.BlockSpec((1, _DC), lambda i: (0, 0)),
        ],
        out_specs=pl.BlockSpec((1000, _DC), lambda i: (i, 0)),
        out_shape=jax.ShapeDtypeStruct((_N, _DC), jnp.float32),
    )(x, W1, b1, W2, b2)


def _rsqrt(v):
    i = lax.bitcast_convert_type(v, jnp.int32)
    i = jnp.int32(0x5F3759DF) - lax.shift_right_arithmetic(i, 1)
    y = lax.bitcast_convert_type(i, jnp.float32)
    for _ in range(3):
        y = y * (1.5 - 0.5 * v * y * y)
    return y


_MESH = plsc.VectorSubcoreMesh(core_axis_name="c", subcore_axis_name="s")


@functools.partial(
    pl.kernel,
    mesh=_MESH,
    compiler_params=pltpu.CompilerParams(use_tc_tiling_on_sc=False),
    out_type=(jax.ShapeDtypeStruct((_N * _DC,), jnp.float32),
              jax.ShapeDtypeStruct((_NC, _R, _DC), jnp.float32),
              jax.ShapeDtypeStruct((_NC, _R, _DC), jnp.float32)),
    scratch_types=[
        pltpu.VMEM_SHARED((_R, _DC), jnp.float32),
        pltpu.VMEM_SHARED((_R, _DC), jnp.float32),
        pltpu.VMEM((_NSUB, _C), jnp.int32),
        pltpu.VMEM((_NSUB, _C), jnp.int32),
        pltpu.VMEM((_C, _DC), jnp.float32),
        pltpu.VMEM((_C, _DC), jnp.float32),
        pltpu.VMEM((_RT, _DC), jnp.float32),
        pltpu.VMEM((_RT, _DC), jnp.float32),
        pltpu.VMEM((_RT, _DC), jnp.float32),
        pltpu.VMEM((_RT, _DC), jnp.float32),
        pltpu.VMEM((_RT, _DC), jnp.float32),
        pltpu.VMEM((_RT, _DC), jnp.float32),
        pltpu.VMEM((_RT * _DC,), jnp.float32),
        pltpu.SemaphoreType.DMA,
        pltpu.SemaphoreType.DMA,
        pltpu.SemaphoreType.DMA,
    ],
)
def _appnp(h_hbm, src_hbm, dst_hbm, out_hbm, p1_hbm, p2_hbm, T1, T2, src_v,
           dst_v, ebuf0, ebuf1, rn_nio, rn_h0o, rn_n9, b_t1, b_p, b_g, b_ho,
           semg, sems0, sems1):
    cid = lax.axis_index("c")
    sid = lax.axis_index("s")
    wid = cid * _NT + sid
    ocid = 1 - cid
    r0 = sid * _RT
    f0 = sid * (_RT * _DC)

    zeros16 = jnp.zeros((_DC,), jnp.float32)
    ones16 = jnp.full((_DC,), 1.0, jnp.float32)

    pltpu.sync_copy(src_hbm.at[wid], src_v)
    pltpu.sync_copy(dst_hbm.at[wid], dst_v)

    def _zero_bg(i, carry):
        b_g[i] = zeros16
        return carry

    def _fill_ones(i, carry):
        ebuf0[i] = ones16
        return carry

    lax.fori_loop(0, _RT, _zero_bg, 0)
    lax.fori_loop(0, _C, _fill_ones, 0)

    pltpu.sync_copy(b_g, T1.at[pl.ds(r0, _RT)])
    pltpu.sync_copy(b_g, T2.at[pl.ds(r0, _RT)])
    plsc.subcore_barrier()

    def _deg(j, carry):
        pltpu.async_copy(ebuf0, T1.at[src_v.at[j]], sems0, add=True)
        pltpu.async_copy(ebuf0, T2.at[dst_v.at[j]], sems1, add=True)
        return carry

    lax.fori_loop(0, _NSUB, _deg, 0)

    def _deg_drain(j, carry):
        pltpu.make_async_copy(ebuf0, T1.at[src_v.at[0]], sems0).wait()
        pltpu.make_async_copy(ebuf0, T2.at[dst_v.at[0]], sems1).wait()
        return carry

    lax.fori_loop(0, _NSUB, _deg_drain, 0)
    plsc.subcore_barrier()

    pltpu.sync_copy(T2.at[pl.ds(r0, _RT)], p1_hbm.at[cid, pl.ds(r0, _RT)])
    pltpu.sync_copy(T1.at[pl.ds(r0, _RT)], p2_hbm.at[cid, pl.ds(r0, _RT)])
    plsc.subcore_barrier()

    pltpu.sync_copy(T2.at[pl.ds(r0, _RT)], b_t1)
    pltpu.sync_copy(p1_hbm.at[ocid, pl.ds(r0, _RT)], b_p)

    def _tf1(i, carry):
        ni = _rsqrt(jnp.maximum(b_t1[i] + b_p[i], 1.0))
        rn_n9[i] = (1.0 - _ALPHA) * ni
        return carry

    lax.fori_loop(0, _RT, _tf1, 0)

    pltpu.sync_copy(T1.at[pl.ds(r0, _RT)], b_t1)
    pltpu.sync_copy(p2_hbm.at[ocid, pl.ds(r0, _RT)], b_p)
    pltpu.sync_copy(h_hbm.at[pl.ds(f0, _RT * _DC)], b_ho)

    def _tf2(i, carry):
        no = _rsqrt(jnp.maximum(b_t1[i] + b_p[i], 1.0))
        hh = b_ho[pl.ds(i * _DC, _DC)]
        rn_nio[i] = no * rn_n9[i]
        rn_h0o[i] = _ALPHA * no * hh
        b_t1[i] = no * hh
        return carry

    lax.fori_loop(0, _RT, _tf2, 0)

    pltpu.sync_copy(b_t1, T1.at[pl.ds(r0, _RT)])
    pltpu.sync_copy(b_g, T2.at[pl.ds(r0, _RT)])
    plsc.subcore_barrier()

    def _step(k, carry):
        def _edge2(jj, c2):
            j0 = 2 * jj
            j1 = j0 + 1

            @pl.when(jj > 0)
            def _drain0():
                pltpu.make_async_copy(ebuf0, T2.at[dst_v.at[0]], sems0).wait()

            pltpu.async_copy(T1.at[src_v.at[j0]], ebuf0, semg).wait()
            pltpu.async_copy(ebuf0, T2.at[dst_v.at[j0]], sems0, add=True)

            @pl.when(jj > 0)
            def _drain1():
                pltpu.make_async_copy(ebuf1, T2.at[dst_v.at[0]], sems1).wait()

            pltpu.async_copy(T1.at[src_v.at[j1]], ebuf1, semg).wait()
            pltpu.async_copy(ebuf1, T2.at[dst_v.at[j1]], sems1, add=True)
            return c2

        lax.fori_loop(0, _NSUB // 2, _edge2, 0)
        pltpu.make_async_copy(ebuf0, T2.at[dst_v.at[0]], sems0).wait()
        pltpu.make_async_copy(ebuf1, T2.at[dst_v.at[0]], sems1).wait()
        plsc.subcore_barrier()

        pltpu.sync_copy(T2.at[pl.ds(r0, _RT)], p1_hbm.at[cid, pl.ds(r0, _RT)])
        plsc.subcore_barrier()

        @pl.when(k < _K - 1)
        def _upd():
            pltpu.async_copy(T2.at[pl.ds(r0, _RT)], b_t1, semg).wait()
            pltpu.async_copy(p1_hbm.at[ocid, pl.ds(r0, _RT)], b_p, sems1).wait()
            zh = pltpu.async_copy(b_g, T2.at[pl.ds(r0, _RT)], sems0)

            def _u(i, c2):
                b_t1[i] = (b_t1[i] + b_p[i]) * rn_nio[i] + rn_h0o[i]
                return c2

            lax.fori_loop(0, _RT, _u, 0, unroll=8)
            pltpu.async_copy(b_t1, T1.at[pl.ds(r0, _RT)], sems1).wait()
            zh.wait()

        plsc.subcore_barrier()
        return carry

    lax.fori_loop(0, _K, _step, 0)

    @pl.when(cid == 0)
    def _out():
        pltpu.sync_copy(T2.at[pl.ds(r0, _RT)], b_t1)
        pltpu.sync_copy(p1_hbm.at[1, pl.ds(r0, _RT)], b_p)
        pltpu.sync_copy(h_hbm.at[pl.ds(f0, _RT * _DC)], b_ho)

        def _o(i, carry):
            hh = b_ho[pl.ds(i * _DC, _DC)]
            b_ho[pl.ds(i * _DC, _DC)] = (b_t1[i] + b_p[i]) * rn_n9[i] + _ALPHA * hh
            return carry

        lax.fori_loop(0, _RT, _o, 0)
        pltpu.sync_copy(b_ho, out_hbm.at[pl.ds(f0, _RT * _DC)])


@jax.jit
def kernel(features, edge_index, W1, b1, W2, b2):
    h = _mlp(features, W1, b1.reshape(1, -1), W2, b2.reshape(1, -1))
    src = edge_index[0]
    dst = edge_index[1]
    pad = jnp.full((_EPAD - _E,), _N, jnp.int32)
    src_p = jnp.concatenate([src, pad]).reshape(_NC * _NT, _NSUB, _C)
    dst_p = jnp.concatenate([dst, pad]).reshape(_NC * _NT, _NSUB, _C)
    out, _, _ = _appnp(h.reshape(-1), src_p, dst_p)
    return out.reshape(_N, _DC)
